# baseline (speedup 1.0000x reference)
import functools

import jax
import jax.numpy as jnp
from jax import lax
from jax.experimental import pallas as pl
from jax.experimental.pallas import tpu as pltpu
from jax.experimental.pallas import tpu_sc as plsc

N = 10000
T = 32
D = 128
HID = 128
H = 64

N2 = 10240
BN = 512
NBLK = N2 // BN


def _encoder_body(e_ref, m_ref, g0_ref, g1_ref,
                  wxf, whf, bihf, bhhf,
                  wxb, whb, bihb, bhhb,
                  cwxf, cwhf, cbihf, cbhhf,
                  cwxb, cwhb, cbihb, cbhhb,
                  lng, lnb, gw, gb, clsw, clsb,
                  z0_ref, z1_ref, node_ref,
                  ysf, ysb):
    f32 = jnp.float32

    def gru_step(x, h, wx, wh, bih, bhh):
        gi = jnp.dot(x, wx[:], preferred_element_type=f32) + bih[:]
        gh = jnp.dot(h, wh[:], preferred_element_type=f32) + bhh[:]
        r = jax.nn.sigmoid(gi[:, 0:H] + gh[:, 0:H])
        zz = jax.nn.sigmoid(gi[:, H:2 * H] + gh[:, H:2 * H])
        nn_ = jnp.tanh(gi[:, 2 * H:3 * H] + r * gh[:, 2 * H:3 * H])
        return (1.0 - zz) * nn_ + zz * h

    h0 = jnp.zeros((BN, H), f32)

    def gen_step(it, carry):
        hf, hb = carry
        tb = T - 1 - it
        xf = e_ref[it] * m_ref[it][:, None]
        xb = e_ref[tb] * m_ref[tb][:, None]
        hf = gru_step(xf, hf, wxf, whf, bihf, bhhf)
        hb = gru_step(xb, hb, wxb, whb, bihb, bhhb)
        ysf[it] = hf
        ysb[tb] = hb
        return (hf, hb)

    jax.lax.fori_loop(0, T, gen_step, (h0, h0))

    def rat_step(it, carry):
        go = jnp.concatenate([ysf[it], ysb[it]], axis=-1)
        mu = jnp.mean(go, axis=-1, keepdims=True)
        var = jnp.mean((go - mu) ** 2, axis=-1, keepdims=True)
        ln = (go - mu) / jnp.sqrt(var + 1e-5) * lng[0] + lnb[0]
        gl = jnp.dot(ln, gw[:], preferred_element_type=f32) + gb[:]
        s0 = gl[:, 0] + g0_ref[it]
        s1 = gl[:, 1] + g1_ref[it]
        s = s1 - s0
        ys1 = jax.nn.sigmoid(s)
        ys0 = jax.nn.sigmoid(-s)
        hard1 = (s > 0.0).astype(f32)
        z0_ref[it] = ((1.0 - hard1) - ys0) + ys0
        z1_ref[it] = (hard1 - ys1) + ys1
        return carry

    jax.lax.fori_loop(0, T, rat_step, 0)

    neg = jnp.full((BN, H), -1000000.0, f32)

    def cls_step(it, carry):
        hf, hb, mxf, mxb = carry
        tb = T - 1 - it
        sf = (m_ref[it] * z1_ref[it])[:, None]
        sb = (m_ref[tb] * z1_ref[tb])[:, None]
        xf = e_ref[it] * sf
        xb = e_ref[tb] * sb
        hf = gru_step(xf, hf, cwxf, cwhf, cbihf, cbhhf)
        hb = gru_step(xb, hb, cwxb, cwhb, cbihb, cbhhb)
        mf = m_ref[it][:, None]
        mb = m_ref[tb][:, None]
        mxf = jnp.maximum(mxf, hf * mf + (1.0 - mf) * neg)
        mxb = jnp.maximum(mxb, hb * mb + (1.0 - mb) * neg)
        return (hf, hb, mxf, mxb)

    _, _, mxf, mxb = jax.lax.fori_loop(
        0, T, cls_step, (h0, h0, neg, neg))

    pooled = jnp.concatenate([mxf, mxb], axis=-1)
    node_ref[:] = jnp.dot(pooled, clsw[:],
                          preferred_element_type=f32) + clsb[:]


def _encoder(eT, mT, g0T, g1T, gen_gru, cls_gru, ln_g, ln_b, genfc_W,
             genfc_b, clsfc_W, clsfc_b, interpret=False):
    f32 = jnp.float32
    args = []
    for p in (gen_gru, cls_gru):
        args += [p[0].T, p[1].T, p[2].reshape(1, 3 * H),
                 p[3].reshape(1, 3 * H),
                 p[4].T, p[5].T, p[6].reshape(1, 3 * H),
                 p[7].reshape(1, 3 * H)]
    gw_pad = jnp.zeros((HID, 128), f32).at[:, :2].set(genfc_W.T)
    gb_pad = jnp.zeros((1, 128), f32).at[0, :2].set(genfc_b)
    args += [ln_g.reshape(1, HID), ln_b.reshape(1, HID),
             gw_pad, gb_pad, clsfc_W.T, clsfc_b.reshape(1, HID)]
    wspecs = [pl.BlockSpec(a.shape, lambda j, nd=a.ndim: (0,) * nd)
              for a in args]
    outs = pl.pallas_call(
        _encoder_body,
        grid=(NBLK,),
        in_specs=[
            pl.BlockSpec((T, BN, D), lambda j: (0, j, 0)),
            pl.BlockSpec((T, BN), lambda j: (0, j)),
            pl.BlockSpec((T, BN), lambda j: (0, j)),
            pl.BlockSpec((T, BN), lambda j: (0, j)),
        ] + wspecs,
        out_specs=[
            pl.BlockSpec((T, BN), lambda j: (0, j)),
            pl.BlockSpec((T, BN), lambda j: (0, j)),
            pl.BlockSpec((BN, HID), lambda j: (j, 0)),
        ],
        out_shape=[
            jax.ShapeDtypeStruct((T, N2), f32),
            jax.ShapeDtypeStruct((T, N2), f32),
            jax.ShapeDtypeStruct((N2, HID), f32),
        ],
        scratch_shapes=[
            pltpu.VMEM((T, BN, H), f32),
            pltpu.VMEM((T, BN, H), f32),
        ],
        interpret=interpret,
    )
    return outs(eT, mT, g0T, g1T, *args)


E = 320000
CH = 128
NW = 32
CPW = -(-E // (NW * CH))
E2 = NW * CPW * CH
RPT = N2 // 16


def _sc_mesh():
    return plsc.VectorSubcoreMesh(core_axis_name="c", subcore_axis_name="s")


def _sc_edge_sum(width, gather):
    @functools.partial(
        pl.kernel,
        out_type=jax.ShapeDtypeStruct((2, N2, width), jnp.float32),
        mesh=_sc_mesh(),
        scratch_types=[
            pltpu.VMEM((CPW, CH), jnp.int32),
            pltpu.VMEM((CPW, CH), jnp.int32),
            pltpu.VMEM((CH, width), jnp.float32),
            pltpu.VMEM_SHARED((N2, width), jnp.float32),
            pltpu.SemaphoreType.DMA,
        ],
    )
    def k(src_hbm, dst_hbm, val_hbm, zer_hbm, out_hbm,
          sidx, didx, rows, acc, sem):
        cid = lax.axis_index("c")
        sid = lax.axis_index("s")
        wid = sid * 2 + cid
        pltpu.sync_copy(zer_hbm.at[pl.ds(sid * RPT, RPT)],
                        acc.at[pl.ds(sid * RPT, RPT)])
        pltpu.sync_copy(dst_hbm.at[wid], didx)
        if gather:
            pltpu.sync_copy(src_hbm.at[wid], sidx)
        else:
            pltpu.sync_copy(val_hbm, rows)
        plsc.subcore_barrier()

        def body(c, carry):
            if gather:
                pltpu.async_copy(val_hbm.at[sidx.at[c]], rows, sem).wait()
            pltpu.sync_copy(rows, acc.at[didx.at[c]], add=True)
            return carry

        lax.fori_loop(0, CPW, body, 0)
        plsc.subcore_barrier()
        pltpu.sync_copy(acc.at[pl.ds(sid * RPT, RPT)],
                        out_hbm.at[cid, pl.ds(sid * RPT, RPT)])

    return k


BR = 2048


def _dinv_of(degp):
    deg = degp[0] + degp[1] + 1.0
    return jax.lax.rsqrt(deg[:, 0:1])


def _e1_body(node, degp, g1wt, hs1_ref):
    h1 = jnp.dot(node[:], g1wt[:], preferred_element_type=jnp.float32)
    hs1_ref[:] = _dinv_of(degp[:]) * h1


def _lsm(logits):
    m = jnp.max(logits, axis=-1, keepdims=True)
    ex = jnp.exp(logits - m)
    return (logits - m) - jnp.log(jnp.sum(ex, axis=-1, keepdims=True))


def _e2_body(p1, hs1, degp, g1b, pwt, pb, g2wt, out0_ref, hs2_ref):
    dinv = _dinv_of(degp[:])
    x1 = jax.nn.relu(dinv * (p1[0] + p1[1] + hs1[:]) + g1b[:])
    out0_ref[:] = _lsm(jnp.dot(x1, pwt[:],
                               preferred_element_type=jnp.float32) + pb[:])
    hs2_ref[:] = dinv * jnp.dot(x1, g2wt[:],
                                preferred_element_type=jnp.float32)


def _f_body128(p2, hs2, degp, g2b, out_ref):
    dinv = _dinv_of(degp[:])
    out_ref[:] = _lsm(dinv * (p2[0] + p2[1] + hs2[:]) + g2b[:])


def _f_body(p2, hs2, degp, g2b, out_ref):
    dinv = _dinv_of(degp[:])
    out_ref[:] = _lsm(dinv * (p2[0] + p2[1] + hs2[:]) + g2b[:])


def _row_call(body, n_out, out_widths, args):
    f32 = jnp.float32
    in_specs = []
    for a in args:
        if a.ndim == 3 and a.shape[1] == N2:
            in_specs.append(pl.BlockSpec((2, BR, a.shape[2]),
                                         lambda j: (0, j, 0)))
        elif a.shape[0] == N2:
            in_specs.append(pl.BlockSpec((BR,) + a.shape[1:],
                                         lambda j, nd=a.ndim: (j,)
                                         + (0,) * (nd - 1)))
        else:
            in_specs.append(pl.BlockSpec(a.shape,
                                         lambda j, nd=a.ndim: (0,) * nd))
    return pl.pallas_call(
        body,
        grid=(N2 // BR,),
        in_specs=in_specs,
        out_specs=[pl.BlockSpec((BR, w), lambda j: (j, 0))
                   for w in out_widths],
        out_shape=[jax.ShapeDtypeStruct((N2, w), f32) for w in out_widths],
    )(*args)


def kernel(inputs, masks, edge_index, emb, gen_gru, cls_gru, ln_g, ln_b,
           genfc_W, genfc_b, clsfc_W, clsfc_b, g1_W, g1_b, g2_W, g2_b,
           prob_W, prob_b):
    f32 = jnp.float32
    n = inputs.shape[0]

    inT = jnp.zeros((T, N2), jnp.int32).at[:, :n].set(inputs.T)
    mT = jnp.zeros((T, N2), f32).at[:, :n].set(masks.T)
    eT = emb[inT]

    u = jax.random.uniform(jax.random.key(7), (n, T, 2), f32,
                           1e-6, 1.0 - 1e-6)
    gum = -jnp.log(-jnp.log(u))
    g0T = jnp.zeros((T, N2), f32).at[:, :n].set(gum[:, :, 0].T)
    g1T = jnp.zeros((T, N2), f32).at[:, :n].set(gum[:, :, 1].T)

    z0T, z1T, node_full = _encoder(eT, mT, g0T, g1T, gen_gru, cls_gru,
                                   ln_g, ln_b, genfc_W, genfc_b,
                                   clsfc_W, clsfc_b)

    z = jnp.stack([z0T[:, :n].T, z1T[:, :n].T], axis=-1)

    pad = jnp.full((E2 - E,), N2 - 1, jnp.int32)
    srcp = jnp.concatenate([edge_index[0], pad]).reshape(NW, CPW, CH)
    dstp = jnp.concatenate([edge_index[1], pad]).reshape(NW, CPW, CH)
    ones128 = jnp.ones((CH, 128), f32)
    zer128 = jnp.zeros((N2, 128), f32)

    degp = _sc_edge_sum(128, False)(srcp, dstp, ones128, zer128)

    g1wt = g1_W.T
    pwt = jnp.zeros((HID, 128), f32).at[:, :8].set(prob_W.T)
    pb = jnp.full((1, 128), -1e30, f32).at[0, :8].set(prob_b)
    g2wt = jnp.zeros((HID, 128), f32).at[:, :8].set(g2_W.T)
    g2b = jnp.full((1, 128), -1e30, f32).at[0, :8].set(g2_b)
    g1b = g1_b.reshape(1, HID)

    (hs1,) = _row_call(_e1_body, 1, [128], [node_full, degp, g1wt])
    p1 = _sc_edge_sum(128, True)(srcp, dstp, hs1, zer128)
    out0f, hs2 = _row_call(_e2_body, 2, [128, 128],
                           [p1, hs1, degp, g1b, pwt, pb, g2wt])
    p2 = _sc_edge_sum(128, True)(srcp, dstp, hs2, zer128)
    (outf,) = _row_call(_f_body128, 1, [128], [p2, hs2, degp, g2b])

    return (z, outf[:n, :8], out0f[:n, :8])
